# baseline (speedup 1.0000x reference)
import functools

import jax
import jax.numpy as jnp
from jax import lax
from jax.experimental import pallas as pl
from jax.experimental.pallas import tpu as pltpu
from jax.experimental.pallas import tpu_sc as plsc

B = 16384
NC = 2
NS = 16
NW = NC * NS
BPW = B // NW
CHUNK = 256
NCHUNK = BPW // CHUNK
L = 16


def _sc_gather(ids, tables):
    mesh = plsc.VectorSubcoreMesh(core_axis_name="c", subcore_axis_name="s")

    @functools.partial(
        pl.kernel,
        mesh=mesh,
        out_type=[jax.ShapeDtypeStruct((B * L,), jnp.float32)] * 9,
        scratch_types=[pltpu.VMEM((2, CHUNK), jnp.int32),
                       pltpu.VMEM((CHUNK,), jnp.int32),
                       pltpu.VMEM((CHUNK,), jnp.int32),
                       pltpu.VMEM((2, CHUNK + 1, 128), jnp.float32),
                       pltpu.VMEM((2, CHUNK * L), jnp.float32),
                       pltpu.SemaphoreType.DMA,
                       pltpu.SemaphoreType.DMA],
    )
    def k(*refs):
        id_refs, tabs, outs = refs[0:9], refs[9:18], refs[18:27]
        ids_v, rid0, rid1, rows, obuf, gsem, wsem = refs[27:34]
        rid_bufs = (rid0, rid1)
        wid = lax.axis_index("s") * NC + lax.axis_index("c")
        base = wid * BPW
        steps = [(t, j) for t in range(9) for j in range(NCHUNK)]
        nst = len(steps)

        def fire(s):
            p = s % 2
            t, j = steps[s]
            lo = base + j * CHUNK
            pltpu.sync_copy(id_refs[t].at[pl.ds(lo, CHUNK)], ids_v.at[p])

            rid_v = rid_bufs[p]

            def rid_body(g, c2):
                rid_v[pl.ds(g * L, L)] = ids_v[p, pl.ds(g * L, L)] >> 4
                return c2

            lax.fori_loop(0, CHUNK // L, rid_body, 0)
            return pltpu.async_copy(
                tabs[t].at[rid_v],
                rows.at[p, pl.ds(0, CHUNK), :], gsem)

        gcps, wcps = {}, {}
        gcps[0] = fire(0)
        for s in range(nst):
            p = s % 2
            t, j = steps[s]
            if s + 1 < nst:
                gcps[s + 1] = fire(s + 1)
            gcps[s].wait()
            if s >= 2:
                wcps[s - 2].wait()

            def ext_body(g, c3, p=p):
                wb16 = (ids_v[p, pl.ds(g * L, L)] & 15) * 8
                for l in range(L):
                    i = g * L + l
                    obuf[p, pl.ds(i * L, L)] = rows[p, i, pl.ds(wb16[l], L)]
                return c3

            lax.fori_loop(0, CHUNK // L, ext_body, 0)
            wcps[s] = pltpu.async_copy(
                obuf.at[p],
                outs[t].at[pl.ds((base + j * CHUNK) * L, CHUNK * L)], wsem)
        wcps[nst - 2].wait()
        wcps[nst - 1].wait()

    return k(*ids, *tables)


def _tc_mlp(numt, embts, w1, b1, w2, b2, w3, b3, wo, bo):
    BLK = 2048

    def body(numt_ref, e0, e1, e2, e3, e4, e5, e6, e7, e8,
             w1_ref, b1_ref, w2_ref, b2_ref, w3_ref, b3_ref, wo_ref, bo_ref,
             out_ref):
        embs = [e[...][:5 if i < 2 else 8, :]
                for i, e in enumerate((e0, e1, e2, e3, e4, e5, e6, e7, e8))]
        x = jnp.concatenate([numt_ref[...]] + embs, axis=0)
        h = jnp.dot(w1_ref[...], x, preferred_element_type=jnp.float32)
        h = jnp.maximum(h + b1_ref[...], 0.0)
        h = jnp.maximum(
            jnp.dot(w2_ref[...], h, preferred_element_type=jnp.float32)
            + b2_ref[...], 0.0)
        h = jnp.maximum(
            jnp.dot(w3_ref[...], h, preferred_element_type=jnp.float32)
            + b3_ref[...], 0.0)
        out_ref[...] = (
            jnp.dot(wo_ref[...], h, preferred_element_type=jnp.float32)
            + bo_ref[...])

    full = lambda shape: pl.BlockSpec(shape, lambda i: (0, 0))
    return pl.pallas_call(
        body,
        grid=(B // BLK,),
        in_specs=[pl.BlockSpec((62, BLK), lambda i: (0, i))]
        + [pl.BlockSpec((8, BLK), lambda i: (0, i)) for _ in range(9)]
        + [full((40, 128)), full((40, 1)),
           full((40, 40)), full((40, 1)),
           full((40, 40)), full((40, 1)),
           full((1, 40)), full((1, 1))],
        out_specs=pl.BlockSpec((1, BLK), lambda i: (0, i)),
        out_shape=jax.ShapeDtypeStruct((1, B), jnp.float32),
    )(numt, *embts, w1, b1, w2, b2, w3, b3, wo, bo)


def _tc_repack6(tabts):
    VP = 102400
    VB = VP // 8
    RB = VB // 16

    def body(x0, x1, x2, x3, x4, x5, o0, o1, o2, o3, o4, o5):
        for x, o in zip((x0, x1, x2, x3, x4, x5), (o0, o1, o2, o3, o4, o5)):
            xt = jnp.transpose(x[...])
            o[...] = xt.reshape(RB, 16, 8).reshape(RB, 128)

    padded = tuple(jnp.pad(t, ((0, 0), (0, VP - 100000))) for t in tabts)
    return pl.pallas_call(
        body,
        grid=(8,),
        in_specs=[pl.BlockSpec((8, VB), lambda i: (0, i))] * 6,
        out_specs=[pl.BlockSpec((RB, 128), lambda i: (i, 0))] * 6,
        out_shape=[jax.ShapeDtypeStruct((VP // 16, 128), jnp.float32)] * 6,
    )(*padded)


def _as_rec8(table, vpad):
    v, d = table.shape
    if (v, d) != (vpad, 8):
        table = jnp.pad(table, ((0, vpad - v), (0, 8 - d)))
    return jnp.concatenate([table[j::16, :] for j in range(16)], axis=1)


def kernel(numerical_data, drg_id, aprdrg_id, primary_id, secondary_id,
           third_id, fourth_id, fifth_id, pr1_id, mdc,
           emb_drg, emb_aprdrg, emb_primary, emb_secondary, emb_third,
           emb_fourth, emb_fifth, emb_pr1, emb_mdc,
           fc1_w, fc1_b, fc2_w, fc2_b, fc3_w, fc3_b, out_w, out_b):
    ids = tuple(i.astype(jnp.int32) for i in
                (drg_id, aprdrg_id, primary_id, secondary_id, third_id,
                 fourth_id, fifth_id, pr1_id, mdc))
    tables = (_as_rec8(emb_drg, 1008), _as_rec8(emb_aprdrg, 1008),
              _as_rec8(emb_primary, 100000),
              _as_rec8(emb_secondary, 100000),
              _as_rec8(emb_third, 100000),
              _as_rec8(emb_fourth, 100000),
              _as_rec8(emb_fifth, 100000),
              _as_rec8(emb_pr1, 100000),
              _as_rec8(emb_mdc, 32))
    flat = _sc_gather(ids, tables)
    embts = tuple(f.reshape(B, L)[:, :8].T for f in flat)

    out_t = _tc_mlp(numerical_data.T, embts,
                    fc1_w, fc1_b[:, None], fc2_w, fc2_b[:, None],
                    fc3_w, fc3_b[:, None], out_w, out_b[:, None])
    return out_t.T


# speedup vs baseline: 2.4232x; 2.4232x over previous
import functools

import jax
import jax.numpy as jnp
from jax import lax
from jax.experimental import pallas as pl
from jax.experimental.pallas import tpu as pltpu
from jax.experimental.pallas import tpu_sc as plsc

B = 16384
NC = 2
NS = 16
NW = NC * NS
BPW = B // NW
CHUNK = 256
NCHUNK = BPW // CHUNK
L = 16


def _sc_gather(ids, tables):
    mesh = plsc.VectorSubcoreMesh(core_axis_name="c", subcore_axis_name="s")

    @functools.partial(
        pl.kernel,
        mesh=mesh,
        out_type=[jax.ShapeDtypeStruct((B * L,), jnp.float32)] * 9,
        scratch_types=[pltpu.VMEM((2, CHUNK), jnp.int32),
                       pltpu.VMEM((CHUNK,), jnp.int32),
                       pltpu.VMEM((CHUNK,), jnp.int32),
                       pltpu.VMEM((2, CHUNK + 1, 128), jnp.float32),
                       pltpu.VMEM((2, CHUNK * L), jnp.float32),
                       pltpu.SemaphoreType.DMA,
                       pltpu.SemaphoreType.DMA],
    )
    def k(*refs):
        id_refs, tabs, outs = refs[0:9], refs[9:18], refs[18:27]
        ids_v, rid0, rid1, rows, obuf, gsem, wsem = refs[27:34]
        rid_bufs = (rid0, rid1)
        wid = lax.axis_index("s") * NC + lax.axis_index("c")
        base = wid * BPW
        steps = [(t, j) for t in range(9) for j in range(NCHUNK)]
        nst = len(steps)

        def fire(s):
            p = s % 2
            t, j = steps[s]
            lo = base + j * CHUNK
            pltpu.sync_copy(id_refs[t].at[pl.ds(lo, CHUNK)], ids_v.at[p])

            rid_v = rid_bufs[p]

            def rid_body(g, c2):
                rid_v[pl.ds(g * L, L)] = ids_v[p, pl.ds(g * L, L)] >> 4
                return c2

            lax.fori_loop(0, CHUNK // L, rid_body, 0)
            return pltpu.async_copy(
                tabs[t].at[rid_v],
                rows.at[p, pl.ds(0, CHUNK), :], gsem)

        gcps, wcps = {}, {}
        gcps[0] = fire(0)
        for s in range(nst):
            p = s % 2
            t, j = steps[s]
            if s + 1 < nst:
                gcps[s + 1] = fire(s + 1)
            gcps[s].wait()
            if s >= 2:
                wcps[s - 2].wait()

            def ext_body(g, c3, p=p):
                wb16 = (ids_v[p, pl.ds(g * L, L)] & 15) * 8
                for l in range(L):
                    i = g * L + l
                    obuf[p, pl.ds(i * L, L)] = rows[p, i, pl.ds(wb16[l], L)]
                return c3

            lax.fori_loop(0, CHUNK // L, ext_body, 0)
            wcps[s] = pltpu.async_copy(
                obuf.at[p],
                outs[t].at[pl.ds((base + j * CHUNK) * L, CHUNK * L)], wsem)
        wcps[nst - 2].wait()
        wcps[nst - 1].wait()

    return k(*ids, *tables)


def _tc_mlp(numt, embts, w1, b1, w2, b2, w3, b3, wo, bo):
    BLK = 2048

    def body(numt_ref, e0, e1, e2, e3, e4, e5, e6, e7, e8,
             w1_ref, b1_ref, w2_ref, b2_ref, w3_ref, b3_ref, wo_ref, bo_ref,
             out_ref):
        embs = [e[...][:5 if i < 2 else 8, :]
                for i, e in enumerate((e0, e1, e2, e3, e4, e5, e6, e7, e8))]
        x = jnp.concatenate([numt_ref[...]] + embs, axis=0)
        h = jnp.dot(w1_ref[...], x, preferred_element_type=jnp.float32)
        h = jnp.maximum(h + b1_ref[...], 0.0)
        h = jnp.maximum(
            jnp.dot(w2_ref[...], h, preferred_element_type=jnp.float32)
            + b2_ref[...], 0.0)
        h = jnp.maximum(
            jnp.dot(w3_ref[...], h, preferred_element_type=jnp.float32)
            + b3_ref[...], 0.0)
        out_ref[...] = (
            jnp.dot(wo_ref[...], h, preferred_element_type=jnp.float32)
            + bo_ref[...])

    full = lambda shape: pl.BlockSpec(shape, lambda i: (0, 0))
    return pl.pallas_call(
        body,
        grid=(B // BLK,),
        in_specs=[pl.BlockSpec((62, BLK), lambda i: (0, i))]
        + [pl.BlockSpec((8, BLK), lambda i: (0, i)) for _ in range(9)]
        + [full((40, 128)), full((40, 1)),
           full((40, 40)), full((40, 1)),
           full((40, 40)), full((40, 1)),
           full((1, 40)), full((1, 1))],
        out_specs=pl.BlockSpec((1, BLK), lambda i: (0, i)),
        out_shape=jax.ShapeDtypeStruct((1, B), jnp.float32),
    )(numt, *embts, w1, b1, w2, b2, w3, b3, wo, bo)


def _tc_repack6(tabts):
    VP = 102400
    VB = VP // 8
    RB = VB // 16

    def body(x0, x1, x2, x3, x4, x5, o0, o1, o2, o3, o4, o5):
        for x, o in zip((x0, x1, x2, x3, x4, x5), (o0, o1, o2, o3, o4, o5)):
            xt = jnp.transpose(x[...])
            o[...] = xt.reshape(RB, 16, 8).reshape(RB, 128)

    padded = tuple(jnp.pad(t, ((0, 0), (0, VP - 100000))) for t in tabts)
    return pl.pallas_call(
        body,
        grid=(8,),
        in_specs=[pl.BlockSpec((8, VB), lambda i: (0, i))] * 6,
        out_specs=[pl.BlockSpec((RB, 128), lambda i: (i, 0))] * 6,
        out_shape=[jax.ShapeDtypeStruct((VP // 16, 128), jnp.float32)] * 6,
    )(*padded)


def _as_rec8(table, vpad):
    v, d = table.shape
    if (v, d) != (vpad, 8):
        table = jnp.pad(table, ((0, vpad - v), (0, 8 - d)))
    return (table.T.reshape(8, vpad // 16, 16)
            .transpose(1, 2, 0).reshape(vpad // 16, 128))


def kernel(numerical_data, drg_id, aprdrg_id, primary_id, secondary_id,
           third_id, fourth_id, fifth_id, pr1_id, mdc,
           emb_drg, emb_aprdrg, emb_primary, emb_secondary, emb_third,
           emb_fourth, emb_fifth, emb_pr1, emb_mdc,
           fc1_w, fc1_b, fc2_w, fc2_b, fc3_w, fc3_b, out_w, out_b):
    ids = tuple(i.astype(jnp.int32) for i in
                (drg_id, aprdrg_id, primary_id, secondary_id, third_id,
                 fourth_id, fifth_id, pr1_id, mdc))
    tables = (_as_rec8(emb_drg, 1008), _as_rec8(emb_aprdrg, 1008),
              _as_rec8(emb_primary, 100000),
              _as_rec8(emb_secondary, 100000),
              _as_rec8(emb_third, 100000),
              _as_rec8(emb_fourth, 100000),
              _as_rec8(emb_fifth, 100000),
              _as_rec8(emb_pr1, 100000),
              _as_rec8(emb_mdc, 32))
    flat = _sc_gather(ids, tables)
    embts = tuple(f.reshape(B, L)[:, :8].T for f in flat)

    out_t = _tc_mlp(numerical_data.T, embts,
                    fc1_w, fc1_b[:, None], fc2_w, fc2_b[:, None],
                    fc3_w, fc3_b[:, None], out_w, out_b[:, None])
    return out_t.T
